# baseline (speedup 1.0000x reference)
import functools

import jax
import jax.numpy as jnp
from jax import lax
from jax.experimental import pallas as pl
from jax.experimental.pallas import tpu as pltpu
from jax.experimental.pallas import tpu_sc as plsc

D_EMB = 64
MAXN = 10.0
L = 16
NCORE = 2
NSUB = 16
NWORK = NCORE * NSUB
BAG = 50
GBAGS = 8
GROWS = GBAGS * BAG
NSTREAM = 4
SPG = GROWS // NSTREAM
RSQRT_MAGIC = 0x5F3759DF


def _rsqrt(x):
    i = lax.bitcast_convert_type(x, jnp.int32)
    y = lax.bitcast_convert_type(
        jnp.int32(RSQRT_MAGIC) - lax.shift_right_logical(i, 1), jnp.float32)
    for _ in range(2):
        y = y * (1.5 - 0.5 * x * y * y)
    return y


def _phase(idx_hbm, tab_hbm, res_hbm, wid, idx_v, rowbuf, tbuf, scale_v,
           acc_v, sems, ngrp):
    iota = lax.iota(jnp.int32, L)
    zero = jnp.zeros((L,), jnp.float32)

    pltpu.sync_copy(idx_hbm.at[pl.ds(wid * ngrp * NSTREAM, ngrp * NSTREAM)],
                    idx_v)

    def issue(g, b):
        for j in range(NSTREAM):
            pltpu.async_copy(
                tab_hbm.at[idx_v.at[g * NSTREAM + j]],
                rowbuf.at[b, pl.ds(j * SPG, SPG)],
                sems[b])

    def wait_group(b):
        pltpu.make_async_copy(tab_hbm.at[pl.ds(0, GROWS)], rowbuf.at[b],
                              sems[b]).wait()

    def compute(g, b):
        def blk_body(blk, carry):
            r0 = blk * L
            for j in range(L):
                r = r0 + j
                n2v = None
                for c in range(4):
                    ch = rowbuf[b, r, pl.ds(c * L, L)]
                    n2v = ch * ch if n2v is None else n2v + ch * ch
                plsc.store_scatter(tbuf, [iota * L + j], n2v)
            n2 = tbuf[pl.ds(0, L)]
            for lrow in range(1, L):
                n2 = n2 + tbuf[pl.ds(lrow * L, L)]
            scale = jnp.minimum(1.0, MAXN * _rsqrt(n2))
            scale_v[pl.ds(r0, L)] = scale
            return carry

        lax.fori_loop(0, GROWS // L, blk_body, 0)

        for bag in range(GBAGS):
            for c in range(4):
                acc_v[bag, pl.ds(c * L, L)] = zero

        def acc_body(blk, carry):
            r0 = blk * L
            svec = scale_v[pl.ds(r0, L)]
            for j in range(L):
                r = r0 + j
                s = jnp.take(svec, jnp.full((L,), j, jnp.int32))
                bag = r // BAG
                for c in range(4):
                    ch = rowbuf[b, r, pl.ds(c * L, L)]
                    plsc.addupdate(acc_v.at[bag, pl.ds(c * L, L)], s * ch)
            return carry

        lax.fori_loop(0, GROWS // L, acc_body, 0)

        base_row = wid * (ngrp * GBAGS) + g * GBAGS
        pltpu.sync_copy(acc_v, res_hbm.at[pl.ds(base_row, GBAGS)])

    issue(0, 0)
    issue(1, 1)

    def pair_body(i, carry):
        for b in range(2):
            g = i * 2 + b
            wait_group(b)
            compute(g, b)

            @pl.when(g + 2 < ngrp)
            def _():
                issue(g + 2, b)
        return carry

    lax.fori_loop(0, ngrp // 2, pair_body, 0)


def kernel(input, output, input_table, output_table):
    batch = input.shape[0]
    bags_per_worker = batch // NWORK
    ngrp = bags_per_worker // GBAGS

    in_idx = input.reshape(batch * BAG // SPG, SPG)
    out_idx = output.reshape(batch * BAG // SPG, SPG)

    mesh = plsc.VectorSubcoreMesh(core_axis_name="c", subcore_axis_name="s")

    @functools.partial(
        pl.kernel,
        out_type=(jax.ShapeDtypeStruct((batch, D_EMB), jnp.float32),
                  jax.ShapeDtypeStruct((batch, D_EMB), jnp.float32)),
        mesh=mesh,
        scratch_types=[
            pltpu.VMEM((ngrp * NSTREAM, SPG), jnp.int32),
            pltpu.VMEM((2, GROWS, D_EMB), jnp.float32),
            pltpu.VMEM((L * L,), jnp.float32),
            pltpu.VMEM((GROWS,), jnp.float32),
            pltpu.VMEM((GBAGS, D_EMB), jnp.float32),
            pltpu.SemaphoreType.DMA,
            pltpu.SemaphoreType.DMA,
        ],
        compiler_params=pltpu.CompilerParams(needs_layout_passes=False,
                                             use_tc_tiling_on_sc=False),
    )
    def sc_kernel(in_idx_r, out_idx_r, in_tab_r, out_tab_r,
                  in_res_r, out_res_r,
                  idx_v, rowbuf, tbuf, scale_v, acc_v, sem0, sem1):
        wid = lax.axis_index("s") * NCORE + lax.axis_index("c")
        for idx_hbm, tab_hbm, res_hbm in (
                (in_idx_r, in_tab_r, in_res_r),
                (out_idx_r, out_tab_r, out_res_r)):
            _phase(idx_hbm, tab_hbm, res_hbm, wid, idx_v, rowbuf, tbuf,
                   scale_v, acc_v, (sem0, sem1), ngrp)

    return sc_kernel(in_idx, out_idx, input_table, output_table)


# speedup vs baseline: 1.3150x; 1.3150x over previous
import functools

import jax
import jax.numpy as jnp
from jax import lax
from jax.experimental import pallas as pl
from jax.experimental.pallas import tpu as pltpu
from jax.experimental.pallas import tpu_sc as plsc

D_EMB = 64
MAXN = 10.0
L = 16
NCORE = 2
NSUB = 16
NWORK = NCORE * NSUB
BAG = 50
GBAGS = 8
GROWS = GBAGS * BAG
RSQRT_MAGIC = 0x5F3759DF


def _rsqrt(x):
    i = lax.bitcast_convert_type(x, jnp.int32)
    y = lax.bitcast_convert_type(
        jnp.int32(RSQRT_MAGIC) - lax.shift_right_logical(i, 1), jnp.float32)
    for _ in range(2):
        y = y * (1.5 - 0.5 * x * y * y)
    return y


def _phase(idx_hbm, tab_hbm, res_hbm, dummy_hbm, wid, idx_v, rowbuf, tbuf,
           scale_v, acc_v, sems, ngrp):
    iota = lax.iota(jnp.int32, L)
    zero = jnp.zeros((L,), jnp.float32)
    rows_per_worker = ngrp * GROWS

    pltpu.sync_copy(idx_hbm.at[pl.ds(wid * rows_per_worker, rows_per_worker)],
                    idx_v)

    def issue(g, b):
        base = g * GROWS

        def issue_blk(blk, carry):
            r0 = blk * L
            ivec = idx_v[pl.ds(base + r0, L)]
            for j in range(L):
                i = ivec[j]
                pltpu.async_copy(
                    tab_hbm.at[i],
                    rowbuf.at[b, blk * (L // 2) + j // 2,
                              pl.ds((j % 2) * D_EMB, D_EMB)],
                    sems[b])
            return carry

        lax.fori_loop(0, GROWS // L, issue_blk, 0)

    def wait_group(b):
        pltpu.make_async_copy(dummy_hbm, rowbuf.at[b], sems[b]).wait()

    def _chunk(b, r2, j, c):
        return rowbuf[b, r2, pl.ds((j % 2) * D_EMB + c * L, L)]

    def compute(g, b):
        def blk_body(blk, carry):
            for j in range(L):
                r2 = blk * (L // 2) + j // 2
                n2v = None
                for c in range(4):
                    ch = _chunk(b, r2, j, c)
                    n2v = ch * ch if n2v is None else n2v + ch * ch
                plsc.store_scatter(tbuf, [iota * L + j], n2v)
            n2 = tbuf[pl.ds(0, L)]
            for lrow in range(1, L):
                n2 = n2 + tbuf[pl.ds(lrow * L, L)]
            scale = jnp.minimum(1.0, MAXN * _rsqrt(n2))
            scale_v[pl.ds(blk * L, L)] = scale
            return carry

        lax.fori_loop(0, GROWS // L, blk_body, 0)

        for bag in range(GBAGS):
            for c in range(4):
                acc_v[bag, pl.ds(c * L, L)] = zero

        def acc_body(blk, carry):
            svec = scale_v[pl.ds(blk * L, L)]
            for j in range(L):
                r2 = blk * (L // 2) + j // 2
                s = jnp.take(svec, jnp.full((L,), j, jnp.int32))
                bag = (blk * L + j) // BAG
                for c in range(4):
                    plsc.addupdate(acc_v.at[bag, pl.ds(c * L, L)],
                                   s * _chunk(b, r2, j, c))
            return carry

        lax.fori_loop(0, GROWS // L, acc_body, 0)

        base_row = wid * (ngrp * GBAGS) + g * GBAGS
        pltpu.sync_copy(acc_v, res_hbm.at[pl.ds(base_row, GBAGS)])

    issue(0, 0)
    issue(1, 1)

    def pair_body(i, carry):
        for b in range(2):
            g = i * 2 + b
            wait_group(b)
            compute(g, b)

            @pl.when(g + 2 < ngrp)
            def _():
                issue(g + 2, b)
        return carry

    lax.fori_loop(0, ngrp // 2, pair_body, 0)


def kernel(input, output, input_table, output_table):
    batch = input.shape[0]
    bags_per_worker = batch // NWORK
    ngrp = bags_per_worker // GBAGS

    in_idx = input.reshape(batch * BAG)
    out_idx = output.reshape(batch * BAG)

    mesh = plsc.VectorSubcoreMesh(core_axis_name="c", subcore_axis_name="s")

    @functools.partial(
        pl.kernel,
        out_type=(jax.ShapeDtypeStruct((batch, D_EMB), jnp.float32),
                  jax.ShapeDtypeStruct((batch, D_EMB), jnp.float32)),
        mesh=mesh,
        scratch_types=[
            pltpu.VMEM((batch * BAG // NWORK,), jnp.int32),
            pltpu.VMEM((2, GROWS // 2, 2 * D_EMB), jnp.float32),
            pltpu.VMEM((L * L,), jnp.float32),
            pltpu.VMEM((GROWS,), jnp.float32),
            pltpu.VMEM((GBAGS, D_EMB), jnp.float32),
            pltpu.SemaphoreType.DMA,
            pltpu.SemaphoreType.DMA,
        ],
        compiler_params=pltpu.CompilerParams(needs_layout_passes=False,
                                             use_tc_tiling_on_sc=True),
    )
    def sc_kernel(in_idx_r, out_idx_r, in_tab_r, out_tab_r, dummy_r,
                  in_res_r, out_res_r,
                  idx_v, rowbuf, tbuf, scale_v, acc_v, sem0, sem1):
        wid = lax.axis_index("s") * NCORE + lax.axis_index("c")
        for idx_hbm, tab_hbm, res_hbm in (
                (in_idx_r, in_tab_r, in_res_r),
                (out_idx_r, out_tab_r, out_res_r)):
            _phase(idx_hbm, tab_hbm, res_hbm, dummy_r, wid, idx_v, rowbuf,
                   tbuf, scale_v, acc_v, (sem0, sem1), ngrp)

    dummy = jnp.zeros((GROWS // 2, 2 * D_EMB), jnp.float32)
    return sc_kernel(in_idx, out_idx, input_table, output_table, dummy)
